# baseline (speedup 1.0000x reference)
import functools

import jax
import jax.numpy as jnp
from jax import lax
from jax.experimental import pallas as pl
from jax.experimental.pallas import tpu as pltpu
from jax.experimental.pallas import tpu_sc as plsc

BATCH = 1024
HIDDEN = 128
TILE_V = 1024


def _sc_gather(input_ids, emb_table):
    info = plsc.get_sparse_core_info()
    nc, ns = info.num_cores, info.num_subcores
    nw = nc * ns
    b_per_w = BATCH // nw
    mesh = plsc.VectorSubcoreMesh(core_axis_name="c", subcore_axis_name="s")

    @functools.partial(
        pl.kernel,
        mesh=mesh,
        out_type=jax.ShapeDtypeStruct((BATCH, HIDDEN), jnp.float32),
        scratch_types=[
            pltpu.VMEM((b_per_w,), jnp.int32),
            pltpu.VMEM((b_per_w, HIDDEN), jnp.float32),
            pltpu.SemaphoreType.DMA,
        ],
    )
    def gather_k(idx_hbm, table_hbm, out_hbm, idx_v, rows_v, sem):
        wid = lax.axis_index("s") * nc + lax.axis_index("c")
        base = wid * b_per_w
        pltpu.sync_copy(idx_hbm.at[pl.ds(base, b_per_w)], idx_v)
        pltpu.async_copy(table_hbm.at[idx_v], rows_v, sem).wait()
        pltpu.sync_copy(rows_v, out_hbm.at[pl.ds(base, b_per_w)])

    return gather_k(input_ids, emb_table)


def _mm_body(x_ref, w_ref, b_ref, o_ref):
    o_ref[...] = lax.dot_general(
        x_ref[...], w_ref[...],
        dimension_numbers=(((1,), (1,)), ((), ())),
        preferred_element_type=jnp.float32,
    ) + b_ref[...]


def kernel(input_ids, emb_table, W, b):
    ids = input_ids.astype(jnp.int32)
    x = _sc_gather(ids, emb_table)

    vocab = W.shape[0]
    grid = (vocab + TILE_V - 1) // TILE_V
    b2 = b.reshape(1, vocab)
    logits = pl.pallas_call(
        _mm_body,
        grid=(grid,),
        in_specs=[
            pl.BlockSpec((BATCH, HIDDEN), lambda i: (0, 0)),
            pl.BlockSpec((TILE_V, HIDDEN), lambda i: (i, 0)),
            pl.BlockSpec((1, TILE_V), lambda i: (0, i)),
        ],
        out_specs=pl.BlockSpec((BATCH, TILE_V), lambda i: (0, i)),
        out_shape=jax.ShapeDtypeStruct((BATCH, vocab), jnp.float32),
    )(x, W, b2)
    return logits


# speedup vs baseline: 1.1800x; 1.1800x over previous
import functools

import jax
import jax.numpy as jnp
from jax import lax
from jax.experimental import pallas as pl
from jax.experimental.pallas import tpu as pltpu
from jax.experimental.pallas import tpu_sc as plsc

BATCH = 1024
HIDDEN = 128
TILE_V = 2048
NBUF = 4


def _sc_gather(input_ids, emb_table):
    info = plsc.get_sparse_core_info()
    nc, ns = info.num_cores, info.num_subcores
    nw = nc * ns
    b_per_w = BATCH // nw
    mesh = plsc.VectorSubcoreMesh(core_axis_name="c", subcore_axis_name="s")

    @functools.partial(
        pl.kernel,
        mesh=mesh,
        out_type=jax.ShapeDtypeStruct((BATCH, HIDDEN), jnp.float32),
        scratch_types=[
            pltpu.VMEM((b_per_w,), jnp.int32),
            pltpu.VMEM((b_per_w, HIDDEN), jnp.float32),
            pltpu.SemaphoreType.DMA,
        ],
    )
    def gather_k(idx_hbm, table_hbm, out_hbm, idx_v, rows_v, sem):
        wid = lax.axis_index("s") * nc + lax.axis_index("c")
        base = wid * b_per_w
        pltpu.sync_copy(idx_hbm.at[pl.ds(base, b_per_w)], idx_v)
        pltpu.async_copy(table_hbm.at[idx_v], rows_v, sem).wait()
        pltpu.sync_copy(rows_v, out_hbm.at[pl.ds(base, b_per_w)])

    return gather_k(input_ids, emb_table)


def _mm_body(n_full, tail, grid, x_ref, w_ref, b_ref, out_hbm, strip_ref,
             obuf, sems):
    i = pl.program_id(0)
    slot = lax.rem(i, NBUF)

    @pl.when(jnp.logical_and(i >= NBUF, i < n_full))
    def _():
        pltpu.make_async_copy(
            obuf.at[slot], out_hbm.at[:, pl.ds(0, TILE_V)], sems.at[slot]
        ).wait()

    o = lax.dot_general(
        x_ref[...], w_ref[...],
        dimension_numbers=(((1,), (1,)), ((), ())),
        preferred_element_type=jnp.float32,
    ) + b_ref[...]

    @pl.when(i < n_full)
    def _():
        obuf.at[slot][...] = o
        pltpu.make_async_copy(
            obuf.at[slot],
            out_hbm.at[:, pl.ds(i * TILE_V, TILE_V)],
            sems.at[slot],
        ).start()

    if tail:
        @pl.when(i == grid - 1)
        def _():
            strip_ref[...] = o[:, :tail]

    @pl.when(i == grid - 1)
    def _():
        for j in range(max(n_full - NBUF, 0), n_full):
            pltpu.make_async_copy(
                obuf.at[j % NBUF],
                out_hbm.at[:, pl.ds(0, TILE_V)],
                sems.at[j % NBUF],
            ).wait()


def kernel(input_ids, emb_table, W, b):
    ids = input_ids.astype(jnp.int32)
    x = _sc_gather(ids, emb_table)

    vocab = W.shape[0]
    n_full = vocab // TILE_V
    tail = vocab % TILE_V
    grid = n_full + (1 if tail else 0)
    b2 = b.reshape(1, vocab)

    out_shapes = [jax.ShapeDtypeStruct((BATCH, vocab), jnp.float32)]
    out_specs = [pl.BlockSpec(memory_space=pl.ANY)]
    if tail:
        out_shapes.append(jax.ShapeDtypeStruct((BATCH, tail), jnp.float32))
        out_specs.append(pl.BlockSpec((BATCH, tail), lambda i: (0, 0)))

    outs = pl.pallas_call(
        functools.partial(_mm_body, n_full, tail, grid),
        grid=(grid,),
        in_specs=[
            pl.BlockSpec((BATCH, HIDDEN), lambda i: (0, 0)),
            pl.BlockSpec((TILE_V, HIDDEN), lambda i: (i, 0)),
            pl.BlockSpec((1, TILE_V), lambda i: (0, i)),
        ],
        out_specs=out_specs,
        out_shape=out_shapes,
        scratch_shapes=[
            pltpu.VMEM((NBUF, BATCH, TILE_V), jnp.float32),
            pltpu.SemaphoreType.DMA((NBUF,)),
        ],
    )(x, W, b2)

    if tail:
        main, strip = outs
        return lax.dynamic_update_slice(main, strip, (0, n_full * TILE_V))
    return outs[0]
